# Initial kernel scaffold; baseline (speedup 1.0000x reference)
import jax
import jax.numpy as jnp
from jax.experimental import pallas as pl


def kernel(hidden_states, weight):
    raise NotImplementedError("write your pallas kernel here")


# speedup vs baseline: 1.2953x; 1.2953x over previous
import jax
import jax.numpy as jnp
from jax import lax
from jax.experimental import pallas as pl

TOPK = 8
NE = 64
H = 1024
BT = 2048


def _gate_kernel(x_ref, w_ref, idx_ref, wgt_ref):
    x = x_ref[...]
    w = w_ref[...]
    logits = lax.dot_general(
        x, w, (((1,), (1,)), ((), ())), preferred_element_type=jnp.float32
    )
    m = jnp.max(logits, axis=-1, keepdims=True)
    iota = lax.broadcasted_iota(jnp.int32, logits.shape, 1)
    work = logits
    vals = []
    idxs = []
    for _ in range(TOPK):
        mk = jnp.max(work, axis=-1, keepdims=True)
        cand = jnp.where(work == mk, iota, NE)
        ik = jnp.min(cand, axis=-1, keepdims=True)
        vals.append(mk)
        idxs.append(ik)
        work = jnp.where(iota == ik, -jnp.inf, work)
    v = jnp.concatenate(vals, axis=1)
    e = jnp.exp(v - m)
    s = jnp.sum(e, axis=-1, keepdims=True)
    idx_ref[...] = jnp.concatenate(idxs, axis=1)
    wgt_ref[...] = e / s


def kernel(hidden_states, weight):
    x = hidden_states.reshape(-1, hidden_states.shape[-1])
    t = x.shape[0]
    idx, wgt = pl.pallas_call(
        _gate_kernel,
        grid=(t // BT,),
        in_specs=[
            pl.BlockSpec((BT, H), lambda i: (i, 0)),
            pl.BlockSpec((NE, H), lambda i: (0, 0)),
        ],
        out_specs=[
            pl.BlockSpec((BT, TOPK), lambda i: (i, 0)),
            pl.BlockSpec((BT, TOPK), lambda i: (i, 0)),
        ],
        out_shape=[
            jax.ShapeDtypeStruct((t, TOPK), jnp.int32),
            jax.ShapeDtypeStruct((t, TOPK), jnp.float32),
        ],
    )(x, weight)
    return idx, wgt


# speedup vs baseline: 1.8087x; 1.3963x over previous
import functools

import jax
import jax.numpy as jnp
from jax import lax
from jax.experimental import pallas as pl
from jax.experimental.pallas import tpu as pltpu
from jax.experimental.pallas import tpu_sc as plsc

TOPK = 8
NE = 64
H = 1024
NC = 2
NS = 16
NW = NC * NS
TPW = 1024


def _logits_kernel(x_ref, w_ref, out_ref):
    out_ref[0] = lax.dot_general(
        w_ref[...], x_ref[...], (((1,), (1,)), ((), ())),
        preferred_element_type=jnp.float32,
    )


def _tc_logits(x, weight, nblk):
    return pl.pallas_call(
        _logits_kernel,
        grid=(nblk,),
        in_specs=[
            pl.BlockSpec((TPW, H), lambda i: (i, 0)),
            pl.BlockSpec((NE, H), lambda i: (0, 0)),
        ],
        out_specs=pl.BlockSpec((1, NE, TPW), lambda i: (i, 0, 0)),
        out_shape=jax.ShapeDtypeStruct((nblk, NE, TPW), jnp.float32),
    )(x, weight)


def _sc_topk_body(lg_hbm, idx_hbm, wgt_hbm, lg_v, idx_v, wgt_v):
    wid = lax.axis_index("s") * NC + lax.axis_index("c")
    pltpu.sync_copy(lg_hbm.at[wid], lg_v)

    def group(g, carry):
        t0 = pl.multiple_of(g * 16, 16)
        neg = jnp.full((16,), -jnp.inf, jnp.float32)
        zero = jnp.zeros((16,), jnp.int32)
        vs = [neg] * TOPK
        ixs = [zero] * TOPK
        for e in range(NE):
            x = lg_v[e, pl.ds(t0, 16)]
            ev = jnp.full((16,), e, jnp.int32)
            gt = [x > vs[j] for j in range(TOPK)]
            nv = [None] * TOPK
            ni = [None] * TOPK
            nv[0] = jnp.where(gt[0], x, vs[0])
            ni[0] = jnp.where(gt[0], ev, ixs[0])
            for j in range(1, TOPK):
                nv[j] = jnp.where(gt[j], jnp.where(gt[j - 1], vs[j - 1], x), vs[j])
                ni[j] = jnp.where(gt[j], jnp.where(gt[j - 1], ixs[j - 1], ev), ixs[j])
            vs, ixs = nv, ni
        m = vs[0]
        es = [jnp.exp(vs[k] - m) for k in range(TOPK)]
        s = es[0]
        for k in range(1, TOPK):
            s = s + es[k]
        r = 1.0 / s
        for k in range(TOPK):
            idx_v[k, pl.ds(t0, 16)] = ixs[k]
            wgt_v[k, pl.ds(t0, 16)] = es[k] * r
        return carry

    lax.fori_loop(0, TPW // 16, group, 0)
    pltpu.sync_copy(idx_v, idx_hbm.at[wid])
    pltpu.sync_copy(wgt_v, wgt_hbm.at[wid])


def _sc_topk(logits3, nblk):
    mesh = plsc.VectorSubcoreMesh(core_axis_name="c", subcore_axis_name="s")
    fn = pl.kernel(
        _sc_topk_body,
        out_type=[
            jax.ShapeDtypeStruct((nblk, TOPK, TPW), jnp.int32),
            jax.ShapeDtypeStruct((nblk, TOPK, TPW), jnp.float32),
        ],
        mesh=mesh,
        scratch_types=[
            pltpu.VMEM((NE, TPW), jnp.float32),
            pltpu.VMEM((TOPK, TPW), jnp.int32),
            pltpu.VMEM((TOPK, TPW), jnp.float32),
        ],
    )
    return fn(logits3)


def kernel(hidden_states, weight):
    x = hidden_states.reshape(-1, hidden_states.shape[-1])
    t = x.shape[0]
    nblk = t // TPW
    logits3 = _tc_logits(x, weight, nblk)
    idx3, wgt3 = _sc_topk(logits3, nblk)
    idx = idx3.transpose(0, 2, 1).reshape(t, TOPK)
    wgt = wgt3.transpose(0, 2, 1).reshape(t, TOPK)
    return idx, wgt
